# baseline (speedup 1.0000x reference)
import jax
import jax.numpy as jnp
from jax import lax
from jax.experimental import pallas as pl
from jax.experimental.pallas import tpu as pltpu
from jax.experimental.pallas import tpu_sc as plsc

_B, _L, _D = 4096, 200, 60
_N = _B * _L
_POS = 62
_NC, _NS = 2, 16
_NW = _NC * _NS
_PER_W = _N // _NW
_C = 256
_CHUNKS = _PER_W // _C
_GB = 128
_NB = _C // _GB
_NR = _N // _GB
_PW = 64
_CD = _C * _D


def _body(wt_hbm, w_hbm, h_hbm, t_hbm, ht_hbm, tt_hbm, out_hbm,
          wi_v, hi_v, ti_v, rowsg_v, outc_v, htx_v, ttx_v, semg, semo,
          semi):
    wid = lax.axis_index("s") * _NC + lax.axis_index("c")
    iot = lax.iota(jnp.int32, 16)
    izero16 = jnp.zeros((16,), jnp.int32)

    def idx_row_off(c):
        return (wid * _CHUNKS + c) * _NB

    def idx_descs(nb, c):
        ro = idx_row_off(c)
        return [
            pltpu.make_async_copy(w_hbm.at[pl.ds(ro, _NB)], wi_v.at[nb],
                                  semi),
            pltpu.make_async_copy(h_hbm.at[pl.ds(ro, _NB)], hi_v.at[nb],
                                  semi),
            pltpu.make_async_copy(t_hbm.at[pl.ds(ro, _NB)], ti_v.at[nb],
                                  semi),
        ]

    def copy_idx(nb, c):
        for dsc in idx_descs(nb, c):
            dsc.start()
        for dsc in idx_descs(nb, c):
            dsc.wait()

    def remap(nb):
        for j in range(_NB):
            @plsc.parallel_loop(0, _GB // 16)
            def _rm(q, j=j):
                s = pl.ds(q * 16, 16)
                w16 = wi_v[nb, j, s]
                h16 = hi_v[nb, j, s]
                hi_v[nb, j, s] = jnp.where(w16 == 0, h16 + _POS, h16)

    def gather_descs(nb):
        return [
            pltpu.make_async_copy(
                wt_hbm.at[wi_v.at[nb].at[j]],
                rowsg_v.at[nb].at[pl.ds(j * _GB, _GB)], semg)
            for j in range(_NB)
        ]

    def fire_gathers(nb):
        for dsc in gather_descs(nb):
            dsc.start()

    def wait_gathers(nb):
        for dsc in gather_descs(nb):
            dsc.wait()

    def out_desc(nb, c):
        return pltpu.make_async_copy(
            outc_v.at[nb],
            out_hbm.at[pl.ds((wid * _CHUNKS + c) * _CD, _CD)], semo)

    def compute(nb):
        ob = nb % 2
        for j in range(_NB):
            @plsc.parallel_loop(0, _GB // 16)
            def group(q, j=j):
                s = pl.ds(q * 16, 16)
                h16 = hi_v[nb, j, s]
                t16 = ti_v[nb, j, s]
                rv = iot + (j * _GB) + q * 16
                p0 = rv * _D
                ho = h16 * _D
                to = t16 * _D

                @plsc.parallel_loop(0, _D, unroll=20)
                def dstep(dd):
                    dv = jnp.full((16,), dd, jnp.int32)
                    wv = plsc.load_gather(rowsg_v.at[nb], [rv, dv])
                    a = plsc.load_gather(htx_v, [ho + dd])
                    b = plsc.load_gather(ttx_v, [to + dd])
                    plsc.store_scatter(outc_v.at[ob], [p0 + dd], wv + a + b)

    stg = rowsg_v.at[0].at[pl.ds(0, _GB)]
    for k in range(8):
        wi_v[0, 0, pl.ds(16 * k, 16)] = jnp.minimum(iot + 16 * k, _POS - 1)
        wi_v[0, 1, pl.ds(16 * k, 16)] = izero16
    pltpu.async_copy(wt_hbm.at[wi_v.at[0].at[1]], stg, semg).wait()
    w0s = []
    for k in range(4):
        cv = iot + 16 * k
        w0s.append(plsc.load_gather(stg, [izero16, cv], mask=cv < _D))

    pltpu.async_copy(ht_hbm.at[wi_v.at[0].at[0]], stg, semg).wait()

    def build_ht(r, c):
        rv = jnp.full((16,), r, jnp.int32)
        nz = rv != 0
        for k in range(4):
            cv = iot + 16 * k
            msk = cv < _D
            val = plsc.load_gather(stg, [rv, cv], mask=msk)
            val = jnp.where(nz, val, 0.0)
            plsc.store_scatter(htx_v, [rv * _D + cv], val, mask=msk)
            plsc.store_scatter(htx_v, [(rv + _POS) * _D + cv],
                               val - w0s[k], mask=msk)
        return c
    lax.fori_loop(0, _POS, build_ht, 0)

    pltpu.async_copy(tt_hbm.at[wi_v.at[0].at[0]], stg, semg).wait()

    def build_tt(r, c):
        rv = jnp.full((16,), r, jnp.int32)
        nz = rv != 0
        for k in range(4):
            cv = iot + 16 * k
            msk = cv < _D
            val = plsc.load_gather(stg, [rv, cv], mask=msk)
            val = jnp.where(nz, val, 0.0)
            plsc.store_scatter(ttx_v, [rv * _D + cv], val, mask=msk)
        return c
    lax.fori_loop(0, _POS, build_tt, 0)

    copy_idx(0, 0)
    remap(0)
    fire_gathers(0)
    for dsc in idx_descs(1, 1):
        dsc.start()
    for dsc in idx_descs(2, 2):
        dsc.start()
    for dsc in idx_descs(1, 1):
        dsc.wait()
    fire_gathers(1)
    remap(1)
    for dsc in idx_descs(3, 3):
        dsc.start()

    def outer(g4, cr):
        for b in range(4):
            c = g4 * 4 + b
            wait_gathers(b)

            @pl.when(c + 2 < _CHUNKS)
            def _prefetch():
                b2 = (b + 2) % 4
                for dsc in idx_descs(b2, c + 2):
                    dsc.wait()
                fire_gathers(b2)
                remap(b2)

            @pl.when(c >= 2)
            def _drain():
                out_desc(b % 2, c - 2).wait()

            compute(b)
            out_desc(b % 2, c).start()

            @pl.when(c + 4 < _CHUNKS)
            def _prefetch_idx():
                for dsc in idx_descs(b, c + 4):
                    dsc.start()
        return cr
    lax.fori_loop(0, _CHUNKS // 4, outer, 0)

    out_desc(0, _CHUNKS - 2).wait()
    out_desc(1, _CHUNKS - 1).wait()


def kernel(word, head, tail, word_table, head_table, tail_table):
    w = word.reshape(_NR, _GB).astype(jnp.int32)
    h = head.reshape(_NR, _GB).astype(jnp.int32)
    t = tail.reshape(_NR, _GB).astype(jnp.int32)
    wtp = jnp.pad(word_table, ((0, 0), (0, _PW - _D)))
    htp = jnp.pad(head_table, ((0, 0), (0, _PW - _D)))
    ttp = jnp.pad(tail_table, ((0, 0), (0, _PW - _D)))
    mesh = plsc.VectorSubcoreMesh(
        core_axis_name="c", subcore_axis_name="s",
        num_cores=_NC, num_subcores=_NS)
    run = pl.kernel(
        _body,
        out_type=jax.ShapeDtypeStruct((_N * _D,), jnp.float32),
        mesh=mesh,
        compiler_params=pltpu.CompilerParams(
            needs_layout_passes=False, use_tc_tiling_on_sc=False),
        scratch_types=[
            pltpu.VMEM((4, _NB, _GB), jnp.int32),
            pltpu.VMEM((4, _NB, _GB), jnp.int32),
            pltpu.VMEM((4, _NB, _GB), jnp.int32),
            pltpu.VMEM((4, _C, _PW), jnp.float32),
            pltpu.VMEM((2, _CD), jnp.float32),
            pltpu.VMEM((2 * _POS * _D,), jnp.float32),
            pltpu.VMEM((_POS * _D,), jnp.float32),
            pltpu.SemaphoreType.DMA,
            pltpu.SemaphoreType.DMA,
            pltpu.SemaphoreType.DMA,
        ],
    )
    out = run(wtp, w, h, t, htp, ttp)
    return out.reshape(_B, _L, _D)


# speedup vs baseline: 1.0139x; 1.0139x over previous
import jax
import jax.numpy as jnp
from jax import lax
from jax.experimental import pallas as pl
from jax.experimental.pallas import tpu as pltpu
from jax.experimental.pallas import tpu_sc as plsc

_B, _L, _D = 4096, 200, 60
_N = _B * _L
_POS = 62
_NC, _NS = 2, 16
_NW = _NC * _NS
_PER_W = _N // _NW
_C = 256
_CHUNKS = _PER_W // _C
_GB = 128
_NB = _C // _GB
_NR = _N // _GB
_PW = 64
_CD = _C * _D


def _body(wt_hbm, w_hbm, h_hbm, t_hbm, ht_hbm, tt_hbm, out_hbm,
          wi_v, hi_v, ti_v, rowsg_v, outc_v, htx_v, ttx_v, semg, semo,
          semi):
    wid = lax.axis_index("s") * _NC + lax.axis_index("c")
    iot = lax.iota(jnp.int32, 16)
    izero16 = jnp.zeros((16,), jnp.int32)

    def idx_row_off(c):
        return (wid * _CHUNKS + c) * _NB

    def idx_descs(nb, c):
        ro = idx_row_off(c)
        return [
            pltpu.make_async_copy(w_hbm.at[pl.ds(ro, _NB)], wi_v.at[nb],
                                  semi),
            pltpu.make_async_copy(h_hbm.at[pl.ds(ro, _NB)], hi_v.at[nb],
                                  semi),
            pltpu.make_async_copy(t_hbm.at[pl.ds(ro, _NB)], ti_v.at[nb],
                                  semi),
        ]

    def copy_idx(nb, c):
        for dsc in idx_descs(nb, c):
            dsc.start()
        for dsc in idx_descs(nb, c):
            dsc.wait()

    def remap(nb):
        for j in range(_NB):
            @plsc.parallel_loop(0, _GB // 16)
            def _rm(q, j=j):
                s = pl.ds(q * 16, 16)
                w16 = wi_v[nb, j, s]
                h16 = hi_v[nb, j, s]
                hi_v[nb, j, s] = jnp.where(w16 == 0, h16 + _POS, h16)

    def gather_descs(nb):
        return [
            pltpu.make_async_copy(
                wt_hbm.at[wi_v.at[nb].at[j]],
                rowsg_v.at[nb].at[pl.ds(j * _GB, _GB)], semg)
            for j in range(_NB)
        ]

    def fire_gathers(nb):
        for dsc in gather_descs(nb):
            dsc.start()

    def wait_gathers(nb):
        for dsc in gather_descs(nb):
            dsc.wait()

    def out_desc(nb, c):
        return pltpu.make_async_copy(
            outc_v.at[nb],
            out_hbm.at[pl.ds((wid * _CHUNKS + c) * _CD, _CD)], semo)

    def compute(nb):
        ob = nb % 2
        for j in range(_NB):
            @plsc.parallel_loop(0, _GB // 16)
            def group(q, j=j):
                s = pl.ds(q * 16, 16)
                h16 = hi_v[nb, j, s]
                t16 = ti_v[nb, j, s]
                rv = iot + (j * _GB) + q * 16
                p0 = rv * _D
                ho = h16 * _D
                to = t16 * _D

                @plsc.parallel_loop(0, _D, unroll=12)
                def dstep(dd):
                    dv = jnp.full((16,), dd, jnp.int32)
                    wv = plsc.load_gather(rowsg_v.at[nb], [rv, dv])
                    a = plsc.load_gather(htx_v, [ho + dd])
                    b = plsc.load_gather(ttx_v, [to + dd])
                    plsc.store_scatter(outc_v.at[ob], [p0 + dd], wv + a + b)

    stg = rowsg_v.at[0].at[pl.ds(0, _GB)]
    for k in range(8):
        wi_v[0, 0, pl.ds(16 * k, 16)] = jnp.minimum(iot + 16 * k, _POS - 1)
        wi_v[0, 1, pl.ds(16 * k, 16)] = izero16
    pltpu.async_copy(wt_hbm.at[wi_v.at[0].at[1]], stg, semg).wait()
    w0s = []
    for k in range(4):
        cv = iot + 16 * k
        w0s.append(plsc.load_gather(stg, [izero16, cv], mask=cv < _D))

    pltpu.async_copy(ht_hbm.at[wi_v.at[0].at[0]], stg, semg).wait()

    def build_ht(r, c):
        rv = jnp.full((16,), r, jnp.int32)
        nz = rv != 0
        for k in range(4):
            cv = iot + 16 * k
            msk = cv < _D
            val = plsc.load_gather(stg, [rv, cv], mask=msk)
            val = jnp.where(nz, val, 0.0)
            plsc.store_scatter(htx_v, [rv * _D + cv], val, mask=msk)
            plsc.store_scatter(htx_v, [(rv + _POS) * _D + cv],
                               val - w0s[k], mask=msk)
        return c
    lax.fori_loop(0, _POS, build_ht, 0)

    pltpu.async_copy(tt_hbm.at[wi_v.at[0].at[0]], stg, semg).wait()

    def build_tt(r, c):
        rv = jnp.full((16,), r, jnp.int32)
        nz = rv != 0
        for k in range(4):
            cv = iot + 16 * k
            msk = cv < _D
            val = plsc.load_gather(stg, [rv, cv], mask=msk)
            val = jnp.where(nz, val, 0.0)
            plsc.store_scatter(ttx_v, [rv * _D + cv], val, mask=msk)
        return c
    lax.fori_loop(0, _POS, build_tt, 0)

    copy_idx(0, 0)
    remap(0)
    fire_gathers(0)
    for dsc in idx_descs(1, 1):
        dsc.start()
    for dsc in idx_descs(2, 2):
        dsc.start()
    for dsc in idx_descs(1, 1):
        dsc.wait()
    fire_gathers(1)
    remap(1)
    for dsc in idx_descs(3, 3):
        dsc.start()

    def outer(g4, cr):
        for b in range(4):
            c = g4 * 4 + b
            wait_gathers(b)

            @pl.when(c + 2 < _CHUNKS)
            def _prefetch():
                b2 = (b + 2) % 4
                for dsc in idx_descs(b2, c + 2):
                    dsc.wait()
                fire_gathers(b2)
                remap(b2)

            @pl.when(c >= 2)
            def _drain():
                out_desc(b % 2, c - 2).wait()

            compute(b)
            out_desc(b % 2, c).start()

            @pl.when(c + 4 < _CHUNKS)
            def _prefetch_idx():
                for dsc in idx_descs(b, c + 4):
                    dsc.start()
        return cr
    lax.fori_loop(0, _CHUNKS // 4, outer, 0)

    out_desc(0, _CHUNKS - 2).wait()
    out_desc(1, _CHUNKS - 1).wait()


def kernel(word, head, tail, word_table, head_table, tail_table):
    w = word.reshape(_NR, _GB).astype(jnp.int32)
    h = head.reshape(_NR, _GB).astype(jnp.int32)
    t = tail.reshape(_NR, _GB).astype(jnp.int32)
    wtp = jnp.pad(word_table, ((0, 0), (0, _PW - _D)))
    htp = jnp.pad(head_table, ((0, 0), (0, _PW - _D)))
    ttp = jnp.pad(tail_table, ((0, 0), (0, _PW - _D)))
    mesh = plsc.VectorSubcoreMesh(
        core_axis_name="c", subcore_axis_name="s",
        num_cores=_NC, num_subcores=_NS)
    run = pl.kernel(
        _body,
        out_type=jax.ShapeDtypeStruct((_N * _D,), jnp.float32),
        mesh=mesh,
        compiler_params=pltpu.CompilerParams(
            needs_layout_passes=False, use_tc_tiling_on_sc=False),
        scratch_types=[
            pltpu.VMEM((4, _NB, _GB), jnp.int32),
            pltpu.VMEM((4, _NB, _GB), jnp.int32),
            pltpu.VMEM((4, _NB, _GB), jnp.int32),
            pltpu.VMEM((4, _C, _PW), jnp.float32),
            pltpu.VMEM((2, _CD), jnp.float32),
            pltpu.VMEM((2 * _POS * _D,), jnp.float32),
            pltpu.VMEM((_POS * _D,), jnp.float32),
            pltpu.SemaphoreType.DMA,
            pltpu.SemaphoreType.DMA,
            pltpu.SemaphoreType.DMA,
        ],
    )
    out = run(wtp, w, h, t, htp, ttp)
    return out.reshape(_B, _L, _D)
